# baseline (speedup 1.0000x reference)
import jax
import jax.numpy as jnp
from jax.experimental import pallas as pl
from jax.experimental.pallas import tpu as pltpu

ALPHA_ = 0.5
HB = 16


def _midmax_body(x_ref, o_ref):
    blk = x_ref[0].reshape(HB, 2, 256, 64)
    h0 = blk[:, 0]
    h1 = blk[:, 1]
    vmin = jnp.minimum(h0, h1)
    vmax = jnp.maximum(h0, h1)
    vmin4 = vmin.reshape(HB, 128, 2, 64)
    vmax4 = vmax.reshape(HB, 128, 2, 64)
    m1 = vmin4[:, :, 0, :]
    m2 = vmin4[:, :, 1, :]
    M1 = vmax4[:, :, 0, :]
    M2 = vmax4[:, :, 1, :]
    max4 = jnp.maximum(M1, M2)
    sec = jnp.minimum(jnp.maximum(m1, m2), jnp.minimum(M1, M2))
    o_ref[0] = ALPHA_ * max4 + (1.0 - ALPHA_) * jnp.maximum(sec, 0.0)


def kernel(x):
    B, H, W, C = x.shape
    Ho, Wo = H // 2, W // 2
    grid = (B, Ho // HB)
    out = pl.pallas_call(
        _midmax_body,
        grid=grid,
        in_specs=[pl.BlockSpec((1, 2 * HB, W, C),
                               lambda b, h: (b, h, 0, 0))],
        out_specs=pl.BlockSpec((1, HB, Wo, C), lambda b, h: (b, h, 0, 0)),
        out_shape=jax.ShapeDtypeStruct((B, Ho, Wo, C), x.dtype),
        compiler_params=pltpu.CompilerParams(
            dimension_semantics=("parallel", "arbitrary")),
    )(x)
    return out
